# baseline (device time: 19709 ns/iter reference)
import jax
import jax.numpy as jnp
from jax import lax
from jax.experimental import pallas as pl
from jax.experimental.pallas import tpu as pltpu

N_CHUNKS = 16
ROWS = 1024
CLIP = 5.5
QSCALE = 127.0 / CLIP
DEQ = CLIP / 127.0


def kernel(x, pi):
    rows_per = ROWS // N_CHUNKS

    def body(pi_ref, x_ref, out_ref, send_q, recv_q, send_sems, recv_sems):
        my_x = lax.axis_index("x")
        my_y = lax.axis_index("y")
        my_z = lax.axis_index("z")
        dest = pi_ref[my_x]

        @pl.when(dest == my_x)
        def _():
            out_ref[...] = x_ref[...].astype(jnp.bfloat16)

        @pl.when(dest != my_x)
        def _():
            barrier_sem = pltpu.get_barrier_semaphore()
            pl.semaphore_signal(
                barrier_sem,
                inc=1,
                device_id=(dest, my_y, my_z),
                device_id_type=pl.DeviceIdType.MESH,
            )
            pl.semaphore_wait(barrier_sem, 1)

            rdmas = []
            for k in range(N_CHUNKS):
                rows = pl.ds(k * rows_per, rows_per)
                xq = jnp.clip(x_ref[0, rows, :], -CLIP, CLIP) * QSCALE
                send_q[0, rows, :] = jnp.round(xq).astype(jnp.int8)
                rdma = pltpu.make_async_remote_copy(
                    src_ref=send_q.at[0, rows, :],
                    dst_ref=recv_q.at[0, rows, :],
                    send_sem=send_sems.at[k],
                    recv_sem=recv_sems.at[k],
                    device_id=(dest, my_y, my_z),
                    device_id_type=pl.DeviceIdType.MESH,
                )
                rdma.start()
                rdmas.append(rdma)

            for k in range(N_CHUNKS):
                rows = pl.ds(k * rows_per, rows_per)
                rdmas[k].wait_recv()
                out_ref[0, rows, :] = (
                    recv_q[0, rows, :].astype(jnp.float32) * DEQ
                ).astype(jnp.bfloat16)

            for k in range(N_CHUNKS):
                rdmas[k].wait_send()

    return pl.pallas_call(
        body,
        out_shape=jax.ShapeDtypeStruct(x.shape, jnp.bfloat16),
        in_specs=[
            pl.BlockSpec(memory_space=pltpu.SMEM),
            pl.BlockSpec(memory_space=pltpu.VMEM),
        ],
        out_specs=pl.BlockSpec(memory_space=pltpu.VMEM),
        scratch_shapes=[
            pltpu.VMEM(x.shape, jnp.int8),
            pltpu.VMEM(x.shape, jnp.int8),
            pltpu.SemaphoreType.DMA((N_CHUNKS,)),
            pltpu.SemaphoreType.DMA((N_CHUNKS,)),
        ],
        compiler_params=pltpu.CompilerParams(collective_id=0),
    )(pi, x)


# device time: 19572 ns/iter; 1.0070x vs baseline; 1.0070x over previous
import jax
import jax.numpy as jnp
from jax import lax
from jax.experimental import pallas as pl
from jax.experimental.pallas import tpu as pltpu

N_CHUNKS = 8
ROWS = 1024
CLIP = 5.5
QSCALE = 127.0 / CLIP
DEQ = CLIP / 127.0


def kernel(x, pi):
    rows_per = ROWS // N_CHUNKS

    def body(pi_ref, x_ref, out_ref, send_q, recv_q, send_sems, recv_sems):
        my_x = lax.axis_index("x")
        my_y = lax.axis_index("y")
        my_z = lax.axis_index("z")
        dest = pi_ref[my_x]

        @pl.when(dest == my_x)
        def _():
            out_ref[...] = x_ref[...].astype(jnp.bfloat16)

        @pl.when(dest != my_x)
        def _():
            barrier_sem = pltpu.get_barrier_semaphore()
            pl.semaphore_signal(
                barrier_sem,
                inc=1,
                device_id=(dest, my_y, my_z),
                device_id_type=pl.DeviceIdType.MESH,
            )
            rows0 = pl.ds(0, rows_per)
            xq0 = jnp.clip(x_ref[0, rows0, :], -CLIP, CLIP) * QSCALE
            send_q[0, rows0, :] = jnp.round(xq0).astype(jnp.int8)
            pl.semaphore_wait(barrier_sem, 1)

            rdmas = []
            for k in range(N_CHUNKS):
                rows = pl.ds(k * rows_per, rows_per)
                if k > 0:
                    xq = jnp.clip(x_ref[0, rows, :], -CLIP, CLIP) * QSCALE
                    send_q[0, rows, :] = jnp.round(xq).astype(jnp.int8)
                rdma = pltpu.make_async_remote_copy(
                    src_ref=send_q.at[0, rows, :],
                    dst_ref=recv_q.at[0, rows, :],
                    send_sem=send_sems.at[k],
                    recv_sem=recv_sems.at[k],
                    device_id=(dest, my_y, my_z),
                    device_id_type=pl.DeviceIdType.MESH,
                )
                rdma.start()
                rdmas.append(rdma)

            for k in range(N_CHUNKS):
                rows = pl.ds(k * rows_per, rows_per)
                rdmas[k].wait_recv()
                out_ref[0, rows, :] = (
                    recv_q[0, rows, :].astype(jnp.float32) * DEQ
                ).astype(jnp.bfloat16)

            for k in range(N_CHUNKS):
                rdmas[k].wait_send()

    return pl.pallas_call(
        body,
        out_shape=jax.ShapeDtypeStruct(x.shape, jnp.bfloat16),
        in_specs=[
            pl.BlockSpec(memory_space=pltpu.SMEM),
            pl.BlockSpec(memory_space=pltpu.VMEM),
        ],
        out_specs=pl.BlockSpec(memory_space=pltpu.VMEM),
        scratch_shapes=[
            pltpu.VMEM(x.shape, jnp.int8),
            pltpu.VMEM(x.shape, jnp.int8),
            pltpu.SemaphoreType.DMA((N_CHUNKS,)),
            pltpu.SemaphoreType.DMA((N_CHUNKS,)),
        ],
        compiler_params=pltpu.CompilerParams(collective_id=0),
    )(pi, x)
